# baseline (speedup 1.0000x reference)
import functools

import jax
import jax.numpy as jnp
from jax import lax
from jax.experimental import pallas as pl
from jax.experimental.pallas import tpu as pltpu
from jax.experimental.pallas import tpu_sc as plsc

B = 16384
D = 128
V = 100
VPAD = 128
P = 50

NC, NS = 2, 16
NW = NC * NS

NCHUNK = 2
CB = B // NCHUNK
BPW = CB // NW

BLK = 2048


def _fuse_body(table_ref, wproj_ref, bprop_ref, bproj_ref, wprop_ref,
               ftab_ref, wf_ref):
    wtop = wproj_ref[:D, :]
    wbot = wproj_ref[D:, :]
    bias = bprop_ref[...] @ wbot + bproj_ref[...]
    ftab_ref[...] = jnp.zeros((VPAD, D), jnp.float32)
    ftab_ref[:V, :] = table_ref[...] @ wtop + bias
    wf_ref[...] = wprop_ref[...] @ wbot


def _fuse_weights(table, w_proj, b_prop, b_proj, w_prop):
    return pl.pallas_call(
        _fuse_body,
        out_shape=(
            jax.ShapeDtypeStruct((VPAD, D), jnp.float32),
            jax.ShapeDtypeStruct((P, D), jnp.float32),
        ),
    )(table, w_proj, b_prop.reshape(1, D), b_proj.reshape(1, D), w_prop)


@functools.cache
def _make_sc_gather(chunk):
    mesh = plsc.VectorSubcoreMesh(core_axis_name="c", subcore_axis_name="s",
                                  num_cores=NC, num_subcores=NS)

    @functools.partial(
        pl.kernel,
        out_type=jax.ShapeDtypeStruct((CB, D), jnp.float32),
        mesh=mesh,
        scratch_types=[
            pltpu.VMEM((BPW,), jnp.int32),
            pltpu.VMEM((BPW, D), jnp.float32),
            pltpu.SemaphoreType.DMA,
        ],
    )
    def _sc_gather(ftab_hbm, idx_hbm, out_hbm, idx_v, rows_v, sem):
        wid = lax.axis_index("s") * NC + lax.axis_index("c")
        base = wid * BPW
        pltpu.sync_copy(idx_hbm.at[pl.ds(chunk * CB + base, BPW)], idx_v)
        pltpu.async_copy(ftab_hbm.at[idx_v], rows_v, sem).wait()
        pltpu.sync_copy(rows_v, out_hbm.at[pl.ds(base, BPW)])

    return _sc_gather


def _combine_first_body(wf_ref, g_ref, prop_ref, out_ref):
    out_ref[...] = g_ref[...] + prop_ref[...] @ wf_ref[...]


def _combine_next_body(prev_ref, wf_ref, g_ref, prop_ref, out_ref):
    del prev_ref
    out_ref[...] = g_ref[...] + prop_ref[...] @ wf_ref[...]


def _combine(chunk, w_fused, g, prop, prev):
    grid = CB // BLK
    off = chunk * grid
    wf_spec = pl.BlockSpec((P, D), lambda i: (0, 0))
    g_spec = pl.BlockSpec((BLK, D), lambda i: (i, 0))
    prop_spec = pl.BlockSpec((BLK, P), lambda i: (i + off, 0))
    out_spec = pl.BlockSpec((BLK, D), lambda i: (i + off, 0))
    out_shape = jax.ShapeDtypeStruct((B, D), jnp.float32)
    if prev is None:
        return pl.pallas_call(
            _combine_first_body,
            grid=(grid,),
            in_specs=[wf_spec, g_spec, prop_spec],
            out_specs=out_spec,
            out_shape=out_shape,
        )(w_fused, g, prop)
    return pl.pallas_call(
        _combine_next_body,
        grid=(grid,),
        in_specs=[pl.BlockSpec(memory_space=pl.ANY), wf_spec, g_spec,
                  prop_spec],
        out_specs=out_spec,
        out_shape=out_shape,
        input_output_aliases={0: 0},
    )(prev, w_fused, g, prop)


def kernel(event_type_ids, prop_vectors, event_type_table, W_prop, b_prop,
           W_proj, b_proj):
    ids = event_type_ids.astype(jnp.int32)
    ftab, w_fused = _fuse_weights(event_type_table, W_proj, b_prop, b_proj,
                                  W_prop)
    gathered = [_make_sc_gather(k)(ftab, ids) for k in range(NCHUNK)]
    out = None
    for k in range(NCHUNK):
        out = _combine(k, w_fused, gathered[k], prop_vectors, out)
    return out


# speedup vs baseline: 1.0438x; 1.0438x over previous
import functools

import jax
import jax.numpy as jnp
from jax import lax
from jax.experimental import pallas as pl
from jax.experimental.pallas import tpu as pltpu
from jax.experimental.pallas import tpu_sc as plsc

B = 16384
D = 128
V = 100
P = 50

NC, NS = 2, 16
NW = NC * NS

NCHUNK = 2
CB = B // NCHUNK
BPW = CB // NW

BLK = 2048


def _fuse_body(wproj_ref, bprop_ref, bproj_ref, wprop_ref, wf_ref, bias_ref):
    wbot = wproj_ref[D:, :]
    bias_ref[...] = bprop_ref[...] @ wbot + bproj_ref[...]
    wf_ref[...] = wprop_ref[...] @ wbot


def _fuse_weights(w_proj, b_prop, b_proj, w_prop):
    return pl.pallas_call(
        _fuse_body,
        out_shape=(
            jax.ShapeDtypeStruct((P, D), jnp.float32),
            jax.ShapeDtypeStruct((1, D), jnp.float32),
        ),
    )(w_proj, b_prop.reshape(1, D), b_proj.reshape(1, D), w_prop)


@functools.cache
def _make_sc_gather(chunk):
    mesh = plsc.VectorSubcoreMesh(core_axis_name="c", subcore_axis_name="s",
                                  num_cores=NC, num_subcores=NS)

    @functools.partial(
        pl.kernel,
        out_type=jax.ShapeDtypeStruct((CB, D), jnp.float32),
        mesh=mesh,
        scratch_types=[
            pltpu.VMEM((BPW,), jnp.int32),
            pltpu.VMEM((BPW, D), jnp.float32),
            pltpu.SemaphoreType.DMA,
        ],
    )
    def _sc_gather(table_hbm, idx_hbm, out_hbm, idx_v, rows_v, sem):
        wid = lax.axis_index("s") * NC + lax.axis_index("c")
        base = wid * BPW
        pltpu.sync_copy(idx_hbm.at[pl.ds(chunk * CB + base, BPW)], idx_v)
        pltpu.async_copy(table_hbm.at[idx_v], rows_v, sem).wait()
        pltpu.sync_copy(rows_v, out_hbm.at[pl.ds(base, BPW)])

    return _sc_gather


def _combine_first_body(wtop_ref, wf_ref, bias_ref, g_ref, prop_ref, out_ref):
    out_ref[...] = (g_ref[...] @ wtop_ref[...] + prop_ref[...] @ wf_ref[...]
                    + bias_ref[...])


def _combine_next_body(prev_ref, wtop_ref, wf_ref, bias_ref, g_ref, prop_ref,
                       out_ref):
    del prev_ref
    out_ref[...] = (g_ref[...] @ wtop_ref[...] + prop_ref[...] @ wf_ref[...]
                    + bias_ref[...])


def _combine(chunk, w_proj, w_fused, bias, g, prop, prev):
    grid = CB // BLK
    off = chunk * grid
    wtop_spec = pl.BlockSpec((D, D), lambda i: (0, 0))
    wf_spec = pl.BlockSpec((P, D), lambda i: (0, 0))
    bias_spec = pl.BlockSpec((1, D), lambda i: (0, 0))
    g_spec = pl.BlockSpec((BLK, D), lambda i: (i, 0))
    prop_spec = pl.BlockSpec((BLK, P), lambda i: (i + off, 0))
    out_spec = pl.BlockSpec((BLK, D), lambda i: (i + off, 0))
    out_shape = jax.ShapeDtypeStruct((B, D), jnp.float32)
    if prev is None:
        return pl.pallas_call(
            _combine_first_body,
            grid=(grid,),
            in_specs=[wtop_spec, wf_spec, bias_spec, g_spec, prop_spec],
            out_specs=out_spec,
            out_shape=out_shape,
        )(w_proj, w_fused, bias, g, prop)
    return pl.pallas_call(
        _combine_next_body,
        grid=(grid,),
        in_specs=[pl.BlockSpec(memory_space=pl.ANY), wtop_spec, wf_spec,
                  bias_spec, g_spec, prop_spec],
        out_specs=out_spec,
        out_shape=out_shape,
        input_output_aliases={0: 0},
    )(prev, w_proj, w_fused, bias, g, prop)


def kernel(event_type_ids, prop_vectors, event_type_table, W_prop, b_prop,
           W_proj, b_proj):
    ids = event_type_ids.astype(jnp.int32)
    gathered = [_make_sc_gather(k)(event_type_table, ids)
                for k in range(NCHUNK)]
    w_fused, bias = _fuse_weights(W_proj, b_prop, b_proj, W_prop)
    out = None
    for k in range(NCHUNK):
        out = _combine(k, W_proj, w_fused, bias, gathered[k], prop_vectors,
                       out)
    return out
